# baseline (speedup 1.0000x reference)
import math

import jax
import jax.numpy as jnp
from jax import lax
from jax.experimental import pallas as pl
from jax.experimental.pallas import tpu as pltpu
from jax.experimental.pallas import tpu_sc as plsc

EMBED = 1000000
B = 1024
N = 511
NPAD = 512
K = 3
PER_TREE = N * K
TREES_PER_W = 32
BATCH = 8
NBATCH = TREES_PER_W // BATCH
IDX_BATCH = BATCH * PER_TREE
IDX_BATCH_PAD = 12288
GCHUNKS = IDX_BATCH_PAD // 128
MAP_BATCH = BATCH * N
OUT_BATCH = BATCH * NPAD
LANES = 16

NEG_HALF_LOG_2PI = -0.5 * math.log(2.0 * math.pi)


PLANE = 4096
GTOT = K * PLANE


def _sc_gather_kernel(tp_ref, i0_ref, i1_ref, i2_ref, map_ref,
                      mean_ref, std_ref,
                      idx_v, map_v, rp_v, out_m, out_s, sem):
    info = plsc.get_sparse_core_info()
    nc = info.num_cores
    wid = lax.axis_index("s") * nc + lax.axis_index("c")
    iota = lax.iota(jnp.int32, LANES)
    zeros_i = jnp.zeros((LANES,), jnp.int32)
    tail_mask = iota < 8
    himask = jnp.full((LANES,), -65536, jnp.int32)

    gsems = (sem.at[0], sem.at[1])
    ssem = sem.at[2]

    def zero_tails(bp):
        for k in range(K):
            plsc.store_scatter(idx_v, [bp * GTOT + k * PLANE + MAP_BATCH + iota],
                               zeros_i, mask=tail_mask)
        plsc.store_scatter(map_v, [bp * PLANE + MAP_BATCH + iota],
                           zeros_i, mask=tail_mask)

    def stage(q, bp):
        tstart = wid * TREES_PER_W + q * BATCH
        src = pl.ds(tstart * N, MAP_BATCH)
        cps = []
        for k, ref in enumerate((i0_ref, i1_ref, i2_ref)):
            cps.append(pltpu.async_copy(
                ref.at[src],
                idx_v.at[pl.ds(bp * GTOT + k * PLANE, MAP_BATCH)], ssem))
        cps.append(pltpu.async_copy(
            map_ref.at[src], map_v.at[pl.ds(bp * PLANE, MAP_BATCH)], ssem))
        return cps

    def fire(bp):
        base = bp * GTOT

        def go(s, c):
            for jj in range(8):
                off = base + (s * 8 + jj) * 128
                pltpu.async_copy(tp_ref.at[idx_v.at[pl.ds(off, 128)]],
                                 rp_v.at[pl.ds(off, 128)], gsems[bp])
            return c
        lax.fori_loop(0, GTOT // (8 * 128), go, 0)

    def drain(bp):
        pltpu.make_async_copy(tp_ref.at[pl.ds(0, GTOT)],
                              rp_v.at[pl.ds(bp * GTOT, GTOT)],
                              gsems[bp]).wait()

    def sum_batch(q, bp):
        tstart = wid * TREES_PER_W + q * BATCH

        def per_tree(i, c):
            mbase = i * N
            obase = i * NPAD
            for j16 in range(NPAD // LANES):
                m = plsc.load_gather(
                    map_v, [bp * PLANE + mbase + j16 * LANES + iota])
                p = bp * GTOT + mbase + m
                acc0 = jnp.zeros((LANES,), jnp.float32)
                acc1 = jnp.zeros((LANES,), jnp.float32)
                for k in range(K):
                    v = plsc.load_gather(rp_v, [k * PLANE + p])
                    acc0 = acc0 + plsc.bitcast(lax.shift_left(v, 16),
                                               jnp.float32)
                    acc1 = acc1 + plsc.bitcast(lax.bitwise_and(v, himask),
                                               jnp.float32)
                pos = obase + j16 * LANES + iota
                plsc.store_scatter(out_m, [pos], acc0)
                plsc.store_scatter(out_s, [pos], acc1)
            return c
        lax.fori_loop(0, BATCH, per_tree, 0)
        dst = pl.ds(tstart * NPAD, OUT_BATCH)
        pltpu.sync_copy(out_m, mean_ref.at[dst])
        pltpu.sync_copy(out_s, std_ref.at[dst])

    zero_tails(0)
    for c in stage(0, 0):
        c.wait()
    fire(0)
    for q in range(NBATCH):
        bp = q % 2
        if q + 1 < NBATCH:
            nbp = 1 - bp
            zero_tails(nbp)
            for c in stage(q + 1, nbp):
                c.wait()
            fire(nbp)
        drain(bp)
        sum_batch(q, bp)


def _tc_tail_kernel(mean_ref, std_ref, eps_ref,
                    samp_ref, alpha_ref, logq_ref, t_ref):
    mean = mean_ref[...][:, :N]
    std = std_ref[...][:, :N]
    eps = eps_ref[...]
    col = lax.broadcasted_iota(jnp.int32, (B, N), 1)
    samp = eps * jnp.exp(std) + mean
    samp_ref[...] = samp
    sig = 1.0 / (1.0 + jnp.exp(-(samp - 2.0)))
    alpha_ref[...] = sig[:, :N - 1]
    base = NEG_HALF_LOG_2PI - 0.5 * eps * eps - std
    s1 = jnp.sum(base, axis=1, keepdims=True)
    lgterm = jnp.where(col < N - 1, jnp.log(sig * (1.0 - sig)), 0.0)
    s2 = jnp.sum(lgterm, axis=1, keepdims=True)
    log_t = jnp.sum(jnp.where(col == N - 1, samp, 0.0), axis=1, keepdims=True)
    logq_ref[...] = (s1 - s2 - log_t).reshape(B)
    t_ref[...] = jnp.exp(log_t).reshape(B)


@jax.jit
def kernel(T_alpha, subsplit_idxes, branch_idx_map, eps):
    bits = T_alpha.view(jnp.int32)
    rnd = jnp.int32(0x8000)
    mb = lax.shift_right_logical(bits[:, 0] + rnd, 16)
    sb = lax.bitwise_and(bits[:, 1] + rnd, jnp.int32(-65536))
    tp = jnp.concatenate([lax.bitwise_or(sb, mb),
                          jnp.zeros((8,), jnp.int32)])

    i0 = subsplit_idxes[:, :, 0].reshape(B * N)
    i1 = subsplit_idxes[:, :, 1].reshape(B * N)
    i2 = subsplit_idxes[:, :, 2].reshape(B * N)
    map_flat = branch_idx_map.reshape(B * N)

    mesh = plsc.VectorSubcoreMesh(core_axis_name="c", subcore_axis_name="s")
    mean, std = pl.kernel(
        _sc_gather_kernel,
        out_type=[
            jax.ShapeDtypeStruct((B * NPAD,), jnp.float32),
            jax.ShapeDtypeStruct((B * NPAD,), jnp.float32),
        ],
        mesh=mesh,
        compiler_params=pltpu.CompilerParams(needs_layout_passes=False),
        scratch_types=[
            pltpu.VMEM((2 * GTOT,), jnp.int32),
            pltpu.VMEM((2 * PLANE,), jnp.int32),
            pltpu.VMEM((2 * GTOT,), jnp.int32),
            pltpu.VMEM((OUT_BATCH,), jnp.float32),
            pltpu.VMEM((OUT_BATCH,), jnp.float32),
            pltpu.SemaphoreType.DMA((3,)),
        ],
    )(tp, i0, i1, i2, map_flat)

    samp, alpha_vec, logq, t_out = pl.pallas_call(
        _tc_tail_kernel,
        out_shape=[
            jax.ShapeDtypeStruct((B, N), jnp.float32),
            jax.ShapeDtypeStruct((B, N - 1), jnp.float32),
            jax.ShapeDtypeStruct((B,), jnp.float32),
            jax.ShapeDtypeStruct((B,), jnp.float32),
        ],
    )(mean.reshape(B, NPAD), std.reshape(B, NPAD), eps)

    return (samp, logq, alpha_vec, t_out)


# speedup vs baseline: 1.0210x; 1.0210x over previous
import math

import jax
import jax.numpy as jnp
from jax import lax
from jax.experimental import pallas as pl
from jax.experimental.pallas import tpu as pltpu
from jax.experimental.pallas import tpu_sc as plsc

EMBED = 1000000
B = 1024
N = 511
NPAD = 512
K = 3
PER_TREE = N * K
TREES_PER_W = 32
BATCH = 8
NBATCH = TREES_PER_W // BATCH
IDX_BATCH = BATCH * PER_TREE
IDX_BATCH_PAD = 12288
GCHUNKS = IDX_BATCH_PAD // 128
MAP_BATCH = BATCH * N
OUT_BATCH = BATCH * NPAD
LANES = 16

NEG_HALF_LOG_2PI = -0.5 * math.log(2.0 * math.pi)


PLANE = 4096
GTOT = K * PLANE


def _sc_gather_kernel(tp_ref, i0_ref, i1_ref, i2_ref, map_ref,
                      mean_ref, std_ref,
                      idx_v, map_v, rp_v, out_m, out_s, sem):
    info = plsc.get_sparse_core_info()
    nc = info.num_cores
    wid = lax.axis_index("s") * nc + lax.axis_index("c")
    iota = lax.iota(jnp.int32, LANES)
    zeros_i = jnp.zeros((LANES,), jnp.int32)
    tail_mask = iota < 8
    himask = jnp.full((LANES,), -65536, jnp.int32)

    gsems = (sem.at[0], sem.at[1])
    ssem = sem.at[2]

    def zero_tails(bp):
        for k in range(K):
            plsc.store_scatter(idx_v, [bp * GTOT + k * PLANE + MAP_BATCH + iota],
                               zeros_i, mask=tail_mask)
        plsc.store_scatter(map_v, [bp * PLANE + MAP_BATCH + iota],
                           zeros_i, mask=tail_mask)

    def stage(q, bp):
        tstart = wid * TREES_PER_W + q * BATCH
        src = pl.ds(tstart * N, MAP_BATCH)
        cps = []
        for k, ref in enumerate((i0_ref, i1_ref, i2_ref)):
            cps.append(pltpu.async_copy(
                ref.at[src],
                idx_v.at[pl.ds(bp * GTOT + k * PLANE, MAP_BATCH)], ssem))
        cps.append(pltpu.async_copy(
            map_ref.at[src], map_v.at[pl.ds(bp * PLANE, MAP_BATCH)], ssem))
        return cps

    def fire(bp):
        base = bp * GTOT

        def go(s, c):
            for jj in range(8):
                off = base + (s * 8 + jj) * 128
                pltpu.async_copy(tp_ref.at[idx_v.at[pl.ds(off, 128)]],
                                 rp_v.at[pl.ds(off, 128)], gsems[bp])
            return c
        lax.fori_loop(0, GTOT // (8 * 128), go, 0)

    def drain(bp):
        pltpu.make_async_copy(tp_ref.at[pl.ds(0, GTOT)],
                              rp_v.at[pl.ds(bp * GTOT, GTOT)],
                              gsems[bp]).wait()

    def sum_batch(q, bp):
        tstart = wid * TREES_PER_W + q * BATCH

        def per_tree(i, c):
            mbase = i * N
            obase = i * NPAD
            for j16 in range(NPAD // LANES):
                m = plsc.load_gather(
                    map_v, [bp * PLANE + mbase + j16 * LANES + iota])
                p = bp * GTOT + mbase + m
                acc0 = jnp.zeros((LANES,), jnp.float32)
                acc1 = jnp.zeros((LANES,), jnp.float32)
                for k in range(K):
                    v = plsc.load_gather(rp_v, [k * PLANE + p])
                    acc0 = acc0 + plsc.bitcast(lax.shift_left(v, 16),
                                               jnp.float32)
                    acc1 = acc1 + plsc.bitcast(lax.bitwise_and(v, himask),
                                               jnp.float32)
                pos = obase + j16 * LANES + iota
                plsc.store_scatter(out_m, [pos], acc0)
                plsc.store_scatter(out_s, [pos], acc1)
            return c
        lax.fori_loop(0, BATCH, per_tree, 0)
        dst = pl.ds(tstart * NPAD, OUT_BATCH)
        pltpu.sync_copy(out_m, mean_ref.at[dst])
        pltpu.sync_copy(out_s, std_ref.at[dst])

    zero_tails(0)
    for c in stage(0, 0):
        c.wait()
    fire(0)
    for q in range(NBATCH):
        bp = q % 2
        if q + 1 < NBATCH:
            nbp = 1 - bp
            zero_tails(nbp)
            for c in stage(q + 1, nbp):
                c.wait()
            fire(nbp)
        drain(bp)
        sum_batch(q, bp)


def _tc_tail_kernel(mean_ref, std_ref, eps_ref,
                    samp_ref, alpha_ref, logq_ref, t_ref):
    mean = mean_ref[...][:, :N]
    std = std_ref[...][:, :N]
    eps = eps_ref[...]
    col = lax.broadcasted_iota(jnp.int32, (B, N), 1)
    samp = eps * jnp.exp(std) + mean
    samp_ref[...] = samp
    sig = 1.0 / (1.0 + jnp.exp(-(samp - 2.0)))
    alpha_ref[...] = sig[:, :N - 1]
    base = NEG_HALF_LOG_2PI - 0.5 * eps * eps - std
    s1 = jnp.sum(base, axis=1, keepdims=True)
    lgterm = jnp.where(col < N - 1, jnp.log(sig * (1.0 - sig)), 0.0)
    s2 = jnp.sum(lgterm, axis=1, keepdims=True)
    log_t = jnp.sum(jnp.where(col == N - 1, samp, 0.0), axis=1, keepdims=True)
    logq_ref[...] = s1 - s2 - log_t
    t_ref[...] = jnp.exp(log_t)


@jax.jit
def kernel(T_alpha, subsplit_idxes, branch_idx_map, eps):
    t_pad = jnp.pad(T_alpha, ((0, 1), (0, 0)))
    bits = t_pad.view(jnp.int32)
    rnd = jnp.int32(0x8000)
    mb = lax.shift_right_logical(bits[:, 0] + rnd, 16)
    sb = lax.bitwise_and(bits[:, 1] + rnd, jnp.int32(-65536))
    tp = lax.bitwise_or(sb, mb)

    i0 = subsplit_idxes[:, :, 0].reshape(B * N)
    i1 = subsplit_idxes[:, :, 1].reshape(B * N)
    i2 = subsplit_idxes[:, :, 2].reshape(B * N)
    map_flat = branch_idx_map.reshape(B * N)

    mesh = plsc.VectorSubcoreMesh(core_axis_name="c", subcore_axis_name="s")
    mean, std = pl.kernel(
        _sc_gather_kernel,
        out_type=[
            jax.ShapeDtypeStruct((B * NPAD,), jnp.float32),
            jax.ShapeDtypeStruct((B * NPAD,), jnp.float32),
        ],
        mesh=mesh,
        compiler_params=pltpu.CompilerParams(needs_layout_passes=False),
        scratch_types=[
            pltpu.VMEM((2 * GTOT,), jnp.int32),
            pltpu.VMEM((2 * PLANE,), jnp.int32),
            pltpu.VMEM((2 * GTOT,), jnp.int32),
            pltpu.VMEM((OUT_BATCH,), jnp.float32),
            pltpu.VMEM((OUT_BATCH,), jnp.float32),
            pltpu.SemaphoreType.DMA((3,)),
        ],
    )(tp, i0, i1, i2, map_flat)

    samp, alpha_vec, logq, t_out = pl.pallas_call(
        _tc_tail_kernel,
        out_shape=[
            jax.ShapeDtypeStruct((B, N), jnp.float32),
            jax.ShapeDtypeStruct((B, N - 1), jnp.float32),
            jax.ShapeDtypeStruct((B, 1), jnp.float32),
            jax.ShapeDtypeStruct((B, 1), jnp.float32),
        ],
    )(mean.reshape(B, NPAD), std.reshape(B, NPAD), eps)

    return (samp, logq.reshape(B), alpha_vec, t_out.reshape(B))
